# baseline (speedup 1.0000x reference)
import jax
import jax.numpy as jnp
from jax import lax
from jax.experimental import pallas as pl
from jax.experimental.pallas import tpu as pltpu
from jax.experimental.pallas import tpu_sc as plsc

_NC, _NS, _L = 2, 16, 16
_NW = _NC * _NS
_B = 16384
_D = 32
_BPW = _B // _NW
_G = _BPW // _L
_CBS = 7813 * 1024
_GW = 16
_GRW = _D * _L * _GW


def _body(uids, iids, uembt, iembt, out, uid_v, iid_v, ugran, igran, out_v,
          sem_u, sem_i):
    wid = lax.axis_index("s") * _NC + lax.axis_index("c")
    base = wid * _BPW
    pltpu.sync_copy(uids.at[pl.ds(base, _BPW)], uid_v)
    pltpu.sync_copy(iids.at[pl.ds(base, _BPW)], iid_v)

    iota = lax.iota(jnp.int32, _L)

    def group(g, carry):
        uvec = uid_v[pl.ds(g * _L, _L)]
        ivec = iid_v[pl.ds(g * _L, _L)]
        ub = uvec
        ib = ivec
        ulow = ub & (_GW - 1)
        ilow = ib & (_GW - 1)
        ub16 = ub - ulow
        ib16 = ib - ilow

        def cblk(cb, c2):
            coff = cb * 8 * 1000000
            for ci in range(8):
                up = ub16 + (coff + ci * 1000000)
                ip = ib16 + (coff + ci * 1000000)
                for l in range(_L):
                    row = ((cb * 8 + ci) * _L + l) * _GW
                    pltpu.async_copy(
                        uembt.at[0, pl.ds(pl.multiple_of(up[l], _GW), _GW)],
                        ugran.at[pl.ds(row, _GW)], sem_u)
                    pltpu.async_copy(
                        iembt.at[0, pl.ds(pl.multiple_of(ip[l], _GW), _GW)],
                        igran.at[pl.ds(row, _GW)], sem_i)
            return c2

        lax.fori_loop(0, _D // 8, cblk, 0)
        pltpu.make_async_copy(
            uembt.at[0, pl.ds(0, _GRW)], ugran, sem_u).wait()
        pltpu.make_async_copy(
            iembt.at[0, pl.ds(0, _GRW)], igran, sem_i).wait()

        ubase_idx = iota * _GW + ulow
        ibase_idx = iota * _GW + ilow
        acc = jnp.zeros((_L,), jnp.float32)
        for c in range(_D):
            uv = plsc.load_gather(ugran, [ubase_idx + c * (_L * _GW)])
            iv = plsc.load_gather(igran, [ibase_idx + c * (_L * _GW)])
            acc = acc + uv * iv
        out_v[pl.ds(g * _L, _L)] = acc
        return carry

    lax.fori_loop(0, _G, group, 0)
    pltpu.sync_copy(out_v, out.at[pl.ds(base, _BPW)])


def kernel(user_ids, item_ids, user_emb, item_emb):
    mesh = plsc.VectorSubcoreMesh(
        core_axis_name="c", subcore_axis_name="s",
        num_cores=_NC, num_subcores=_NS)
    f = pl.kernel(
        _body,
        out_type=jax.ShapeDtypeStruct((_B,), jnp.float32),
        mesh=mesh,
        scratch_types=[
            pltpu.VMEM((_BPW,), jnp.int32),
            pltpu.VMEM((_BPW,), jnp.int32),
            pltpu.VMEM((_GRW,), jnp.float32),
            pltpu.VMEM((_GRW,), jnp.float32),
            pltpu.VMEM((_BPW,), jnp.float32),
            pltpu.SemaphoreType.DMA,
            pltpu.SemaphoreType.DMA,
        ],
        compiler_params=pltpu.CompilerParams(
            needs_layout_passes=False, use_tc_tiling_on_sc=False,
            disable_bounds_checks=True),
    )
    return f(user_ids, item_ids, user_emb.T, item_emb.T)


# speedup vs baseline: 1.0070x; 1.0070x over previous
import jax
import jax.numpy as jnp
from jax import lax
from jax.experimental import pallas as pl
from jax.experimental.pallas import tpu as pltpu
from jax.experimental.pallas import tpu_sc as plsc

_NC, _NS, _L = 2, 16, 16
_NW = _NC * _NS
_B = 16384
_D = 32
_BPW = _B // _NW
_G = _BPW // _L
_GW = 16
_RPG = _D * _L
_RPC = 1000000 // _GW


def _body(uids, iids, uembg, iembg, out, uid_v, iid_v,
          ugr0, ugr1, igr0, igr1, uix0, uix1, iix0, iix1, out_v,
          su0, su1, si0, si1):
    wid = lax.axis_index("s") * _NC + lax.axis_index("c")
    base = wid * _BPW
    pltpu.sync_copy(uids.at[pl.ds(base, _BPW)], uid_v)
    pltpu.sync_copy(iids.at[pl.ds(base, _BPW)], iid_v)

    iota = lax.iota(jnp.int32, _L)

    def issue(g, uix, iix, ugr, igr, su, si):
        uvec = uid_v[pl.ds(g * _L, _L)]
        ivec = iid_v[pl.ds(g * _L, _L)]
        ubase = uvec >> 4
        ibase = ivec >> 4
        for c in range(_D):
            uix[pl.ds(c * _L, _L)] = ubase + c * _RPC
            iix[pl.ds(c * _L, _L)] = ibase + c * _RPC
        pltpu.async_copy(uembg.at[uix], ugr, su)
        pltpu.async_copy(iembg.at[iix], igr, si)

    def wait(uix, iix, ugr, igr, su, si):
        pltpu.make_async_copy(uembg.at[uix], ugr, su).wait()
        pltpu.make_async_copy(iembg.at[iix], igr, si).wait()

    def compute(g, ugr, igr):
        uvec = uid_v[pl.ds(g * _L, _L)]
        ivec = iid_v[pl.ds(g * _L, _L)]
        ulow = uvec & (_GW - 1)
        ilow = ivec & (_GW - 1)
        acc = jnp.zeros((_L,), jnp.float32)
        for c in range(_D):
            rows = c * _L + iota
            uv = plsc.load_gather(ugr, [rows, ulow])
            iv = plsc.load_gather(igr, [rows, ilow])
            acc = acc + uv * iv
        out_v[pl.ds(g * _L, _L)] = acc

    issue(0, uix0, iix0, ugr0, igr0, su0, si0)

    def pair(gp, carry):
        g0 = gp * 2
        issue(g0 + 1, uix1, iix1, ugr1, igr1, su1, si1)
        wait(uix0, iix0, ugr0, igr0, su0, si0)
        compute(g0, ugr0, igr0)

        @pl.when(gp < _G // 2 - 1)
        def _():
            issue(g0 + 2, uix0, iix0, ugr0, igr0, su0, si0)

        wait(uix1, iix1, ugr1, igr1, su1, si1)
        compute(g0 + 1, ugr1, igr1)
        return carry

    lax.fori_loop(0, _G // 2, pair, 0)
    pltpu.sync_copy(out_v, out.at[pl.ds(base, _BPW)])


def kernel(user_ids, item_ids, user_emb, item_emb):
    mesh = plsc.VectorSubcoreMesh(
        core_axis_name="c", subcore_axis_name="s",
        num_cores=_NC, num_subcores=_NS)
    f = pl.kernel(
        _body,
        out_type=jax.ShapeDtypeStruct((_B,), jnp.float32),
        mesh=mesh,
        scratch_types=[
            pltpu.VMEM((_BPW,), jnp.int32),
            pltpu.VMEM((_BPW,), jnp.int32),
            pltpu.VMEM((_RPG, _GW), jnp.float32),
            pltpu.VMEM((_RPG, _GW), jnp.float32),
            pltpu.VMEM((_RPG, _GW), jnp.float32),
            pltpu.VMEM((_RPG, _GW), jnp.float32),
            pltpu.VMEM((_RPG,), jnp.int32),
            pltpu.VMEM((_RPG,), jnp.int32),
            pltpu.VMEM((_RPG,), jnp.int32),
            pltpu.VMEM((_RPG,), jnp.int32),
            pltpu.VMEM((_BPW,), jnp.float32),
            pltpu.SemaphoreType.DMA,
            pltpu.SemaphoreType.DMA,
            pltpu.SemaphoreType.DMA,
            pltpu.SemaphoreType.DMA,
        ],
        compiler_params=pltpu.CompilerParams(
            needs_layout_passes=False, use_tc_tiling_on_sc=False,
            disable_bounds_checks=True),
    )
    uembg = user_emb.T.reshape(_D * _RPC, _GW)
    iembg = item_emb.T.reshape(_D * _RPC, _GW)
    return f(user_ids, item_ids, uembg, iembg)
